# baseline (speedup 1.0000x reference)
import jax
import jax.numpy as jnp
from jax import lax
from jax.experimental import pallas as pl
from jax.experimental.pallas import tpu as pltpu
from jax.experimental.pallas import tpu_sc as plsc

B = 16384
T = 512

_LOG2C = (
    5.6422440275483154e-08,
    1.442685851294528,
    -0.7210957682030537,
    0.4781764415123899,
    -0.34542933660333985,
    0.2380419836756127,
    -0.13314692748387624,
    0.04943336843736993,
    -0.008665699320087797,
)
_LN2 = 0.6931471805599453
_LOG2E = 1.4426950408889634

_EXP2C = (
    0.9999980397841518,
    -0.6930489339094811,
    0.23943060364123772,
    -0.05321311778871329,
    0.006835154726653606,
)

_INFO = plsc.get_sparse_core_info()
_NC = _INFO.num_cores
_NS = _INFO.num_subcores
_NW = _NC * _NS
_RW = B // _NW
_CH = 64
_NCHUNK = _RW // _CH


def _treemul(vals):
    while len(vals) > 1:
        vals = [a * b for a, b in zip(vals[::2], vals[1::2])]
    return vals[0]


def _treeadd(vals):
    while len(vals) > 1:
        vals = [a + b for a, b in zip(vals[::2], vals[1::2])]
    return vals[0]


def _sc_kernel(phi_hbm, idx_hbm, ev_hbm, out_hbm,
               buf0, buf1, idx_v, ev_v, stage_v, sem0, sem1):
    wid = lax.axis_index("s") * _NC + lax.axis_index("c")
    base = wid * _RW

    pltpu.sync_copy(idx_hbm.at[pl.ds(base, _RW)], idx_v)
    pltpu.sync_copy(ev_hbm.at[pl.ds(base, _RW)], ev_v)

    bufs = (buf0, buf1)
    sems = (sem0, sem1)

    def start(c):
        return pltpu.async_copy(
            phi_hbm.at[pl.ds((base + c * _CH) * T, _CH * T)],
            bufs[c % 2], sems[c % 2])

    lane = lax.iota(jnp.int32, 16)
    copies = [None] * _NCHUNK
    copies[0] = start(0)
    acc_w = jnp.zeros((16,), jnp.float32)

    for c in range(_NCHUNK):
        if c + 1 < _NCHUNK:
            copies[c + 1] = start(c + 1)
        copies[c].wait()
        buf = bufs[c % 2]

        def group(g, acc_w):
            rowoff = (lane + g * 16) * T
            idx_vec = idx_v[pl.ds(c * _CH + g * 16, 16)]
            ev_vec = ev_v[pl.ds(c * _CH + g * 16, 16)]
            nsteps = jnp.max(idx_vec) // 16 + 1
            colk = [(lane + k) & 15 for k in range(16)]

            init = (jnp.zeros((16,), jnp.float32),
                    jnp.ones((16,), jnp.float32),
                    jnp.zeros((16,), jnp.int32))

            @plsc.parallel_loop(0, nsteps, unroll=2, carry=init)
            def step_out(s, carry):
                S, P, E = carry
                s16 = s * 16
                rem = idx_vec - s16
                base_idx = rowoff + s16
                fs = []
                ss = []
                for k in range(16):
                    x = plsc.load_gather(buf, [base_idx + colk[k]])
                    u = jnp.exp(-jnp.abs(x))
                    m = rem >= colk[k]
                    fs.append(jnp.where(m, 1.0 + u, 1.0))
                    ss.append(jnp.where(m, jnp.maximum(x, 0.0), 0.0))
                P = P * _treemul(fs)
                S = S + _treeadd(ss)
                bits = lax.bitcast_convert_type(P, jnp.int32)
                E = E + lax.shift_right_logical(bits, 23)
                P = lax.bitcast_convert_type(
                    (bits & 0x007FFFFF) | 0x3F800000, jnp.float32)
                return S, P, E

            S, P, E = step_out

            t = P - 1.0
            pl2 = jnp.float32(_LOG2C[-1])
            for coef in _LOG2C[-2::-1]:
                pl2 = coef + t * pl2
            ef = (E - 127 * nsteps).astype(jnp.float32)
            gathered = plsc.load_gather(buf, [rowoff + idx_vec])
            return acc_w + S + _LN2 * (ef + pl2) - ev_vec * gathered

        acc_w = lax.fori_loop(0, _CH // 16, group, acc_w)

    stage_v[...] = acc_w
    pltpu.sync_copy(stage_v, out_hbm.at[wid])


@jax.jit
def _run(phi, idx, ev):
    mesh = plsc.VectorSubcoreMesh(core_axis_name="c", subcore_axis_name="s")
    partials = pl.kernel(
        _sc_kernel,
        mesh=mesh,
        out_type=jax.ShapeDtypeStruct((_NW, 16), jnp.float32),
        scratch_types=[
            pltpu.VMEM((_CH * T,), jnp.float32),
            pltpu.VMEM((_CH * T,), jnp.float32),
            pltpu.VMEM((_RW,), jnp.int32),
            pltpu.VMEM((_RW,), jnp.float32),
            pltpu.VMEM((16,), jnp.float32),
            pltpu.SemaphoreType.DMA,
            pltpu.SemaphoreType.DMA,
        ],
        compiler_params=pltpu.CompilerParams(
            use_tc_tiling_on_sc=False, needs_layout_passes=False),
    )(phi.reshape(-1), idx, ev)
    return jnp.sum(partials) / B


def kernel(phi, idx_durations, events):
    return _run(phi, idx_durations.reshape(-1), events.reshape(-1))


# speedup vs baseline: 1.0541x; 1.0541x over previous
import jax
import jax.numpy as jnp
from jax import lax
from jax.experimental import pallas as pl
from jax.experimental.pallas import tpu as pltpu
from jax.experimental.pallas import tpu_sc as plsc

B = 16384
T = 512

_LOG2C = (
    5.6422440275483154e-08,
    1.442685851294528,
    -0.7210957682030537,
    0.4781764415123899,
    -0.34542933660333985,
    0.2380419836756127,
    -0.13314692748387624,
    0.04943336843736993,
    -0.008665699320087797,
)
_LN2 = 0.6931471805599453
_LOG2E = 1.4426950408889634

_EXP2C = (
    0.9999980397841518,
    -0.6930489339094811,
    0.23943060364123772,
    -0.05321311778871329,
    0.006835154726653606,
)

_INFO = plsc.get_sparse_core_info()
_NC = _INFO.num_cores
_NS = _INFO.num_subcores
_NW = _NC * _NS
_RW = B // _NW
_CH = 64
_NCHUNK = _RW // _CH


def _treemul(vals):
    while len(vals) > 1:
        vals = [a * b for a, b in zip(vals[::2], vals[1::2])]
    return vals[0]


def _treeadd(vals):
    while len(vals) > 1:
        vals = [a + b for a, b in zip(vals[::2], vals[1::2])]
    return vals[0]


def _sc_kernel(phi_hbm, idx_hbm, ev_hbm, out_hbm,
               buf0, buf1, idx_v, ev_v, stage_v, sem0, sem1):
    wid = lax.axis_index("s") * _NC + lax.axis_index("c")
    base = wid * _RW

    pltpu.sync_copy(idx_hbm.at[pl.ds(base, _RW)], idx_v)
    pltpu.sync_copy(ev_hbm.at[pl.ds(base, _RW)], ev_v)

    bufs = (buf0, buf1)
    sems = (sem0, sem1)

    def start(c):
        return pltpu.async_copy(
            phi_hbm.at[pl.ds((base + c * _CH) * T, _CH * T)],
            bufs[c % 2], sems[c % 2])

    lane = lax.iota(jnp.int32, 16)
    copies = [None] * _NCHUNK
    copies[0] = start(0)
    acc_w = jnp.zeros((16,), jnp.float32)

    for c in range(_NCHUNK):
        if c + 1 < _NCHUNK:
            copies[c + 1] = start(c + 1)
        copies[c].wait()
        buf = bufs[c % 2]

        def group(g, acc_w):
            rowoff = (lane + g * 16) * T
            idx_vec = idx_v[pl.ds(c * _CH + g * 16, 16)]
            ev_vec = ev_v[pl.ds(c * _CH + g * 16, 16)]
            nsteps = jnp.max(idx_vec) // 16 + 1
            colk = [(lane + k) & 15 for k in range(16)]

            init = (jnp.zeros((16,), jnp.float32),
                    jnp.ones((16,), jnp.float32),
                    jnp.zeros((16,), jnp.int32))

            @plsc.parallel_loop(0, nsteps, unroll=1, carry=init)
            def step_out(s, carry):
                S, P, E = carry
                s16 = s * 16
                rem = idx_vec - s16
                base_idx = rowoff + s16
                fs = []
                ss = []
                for k in range(16):
                    x = plsc.load_gather(buf, [base_idx + colk[k]])
                    u = jnp.exp(-jnp.abs(x))
                    m = rem >= colk[k]
                    fs.append(jnp.where(m, 1.0 + u, 1.0))
                    ss.append(jnp.where(m, jnp.maximum(x, 0.0), 0.0))
                P = P * _treemul(fs)
                S = S + _treeadd(ss)
                bits = lax.bitcast_convert_type(P, jnp.int32)
                E = E + lax.shift_right_logical(bits, 23)
                P = lax.bitcast_convert_type(
                    (bits & 0x007FFFFF) | 0x3F800000, jnp.float32)
                return S, P, E

            S, P, E = step_out

            t = P - 1.0
            pl2 = jnp.float32(_LOG2C[-1])
            for coef in _LOG2C[-2::-1]:
                pl2 = coef + t * pl2
            ef = (E - 127 * nsteps).astype(jnp.float32)
            gathered = plsc.load_gather(buf, [rowoff + idx_vec])
            return acc_w + S + _LN2 * (ef + pl2) - ev_vec * gathered

        acc_w = lax.fori_loop(0, _CH // 16, group, acc_w)

    stage_v[...] = acc_w
    pltpu.sync_copy(stage_v, out_hbm.at[wid])


@jax.jit
def _run(phi, idx, ev):
    mesh = plsc.VectorSubcoreMesh(core_axis_name="c", subcore_axis_name="s")
    partials = pl.kernel(
        _sc_kernel,
        mesh=mesh,
        out_type=jax.ShapeDtypeStruct((_NW, 16), jnp.float32),
        scratch_types=[
            pltpu.VMEM((_CH * T,), jnp.float32),
            pltpu.VMEM((_CH * T,), jnp.float32),
            pltpu.VMEM((_RW,), jnp.int32),
            pltpu.VMEM((_RW,), jnp.float32),
            pltpu.VMEM((16,), jnp.float32),
            pltpu.SemaphoreType.DMA,
            pltpu.SemaphoreType.DMA,
        ],
        compiler_params=pltpu.CompilerParams(
            use_tc_tiling_on_sc=False, needs_layout_passes=False),
    )(phi.reshape(-1), idx, ev)
    return jnp.sum(partials) / B


def kernel(phi, idx_durations, events):
    return _run(phi, idx_durations.reshape(-1), events.reshape(-1))


# speedup vs baseline: 1.1511x; 1.0920x over previous
import jax
import jax.numpy as jnp
from jax import lax
from jax.experimental import pallas as pl
from jax.experimental.pallas import tpu as pltpu
from jax.experimental.pallas import tpu_sc as plsc

B = 16384
T = 512

_LOG2C = (
    5.6422440275483154e-08,
    1.442685851294528,
    -0.7210957682030537,
    0.4781764415123899,
    -0.34542933660333985,
    0.2380419836756127,
    -0.13314692748387624,
    0.04943336843736993,
    -0.008665699320087797,
)
_LN2 = 0.6931471805599453
_LOG2E = 1.4426950408889634

_EXP2C = (
    0.9999980397841518,
    -0.6930489339094811,
    0.23943060364123772,
    -0.05321311778871329,
    0.006835154726653606,
)

_INFO = plsc.get_sparse_core_info()
_NC = _INFO.num_cores
_NS = _INFO.num_subcores
_NW = _NC * _NS
_RW = B // _NW
_CH = 64
_NCHUNK = _RW // _CH


def _treemul(vals):
    while len(vals) > 1:
        vals = [a * b for a, b in zip(vals[::2], vals[1::2])]
    return vals[0]


def _treeadd(vals):
    while len(vals) > 1:
        vals = [a + b for a, b in zip(vals[::2], vals[1::2])]
    return vals[0]


def _sc_kernel(phi_hbm, idx_hbm, ev_hbm, out_hbm,
               buf0, buf1, idx_v, ev_v, stage_v, sem0, sem1):
    wid = lax.axis_index("s") * _NC + lax.axis_index("c")
    base = wid * _RW

    pltpu.sync_copy(idx_hbm.at[pl.ds(base, _RW)], idx_v)
    pltpu.sync_copy(ev_hbm.at[pl.ds(base, _RW)], ev_v)

    bufs = (buf0, buf1)
    sems = (sem0, sem1)

    def start(c):
        return pltpu.async_copy(
            phi_hbm.at[pl.ds((base + c * _CH) * T, _CH * T)],
            bufs[c % 2], sems[c % 2])

    lane = lax.iota(jnp.int32, 16)
    copies = [None] * _NCHUNK
    copies[0] = start(0)
    acc_w = jnp.zeros((16,), jnp.float32)

    for c in range(_NCHUNK):
        if c + 1 < _NCHUNK:
            copies[c + 1] = start(c + 1)
        copies[c].wait()
        buf = bufs[c % 2]

        def group(g, acc_w):
            rowoff = (lane + g * 16) * T
            idx_vec = idx_v[pl.ds(c * _CH + g * 16, 16)]
            ev_vec = ev_v[pl.ds(c * _CH + g * 16, 16)]
            nsteps = jnp.max(idx_vec) // 16 + 1
            colk = [(lane + k) & 15 for k in range(16)]

            init = (jnp.zeros((16,), jnp.float32),
                    jnp.ones((16,), jnp.float32),
                    jnp.zeros((16,), jnp.int32))

            @plsc.parallel_loop(0, nsteps, unroll=1, carry=init)
            def step_out(s, carry):
                S, P, E = carry
                s16 = s * 16
                rem = idx_vec - s16
                base_idx = rowoff + s16
                fs = []
                ss = []
                for k in range(16):
                    x = plsc.load_gather(buf, [base_idx + colk[k]])
                    xm = jnp.where(rem >= colk[k], x, -1e30)
                    u = jnp.exp(-jnp.abs(xm))
                    fs.append(1.0 + u)
                    ss.append(jnp.maximum(xm, 0.0))
                P = P * _treemul(fs)
                S = S + _treeadd(ss)
                bits = lax.bitcast_convert_type(P, jnp.int32)
                E = E + lax.shift_right_logical(bits, 23)
                P = lax.bitcast_convert_type(
                    (bits & 0x007FFFFF) | 0x3F800000, jnp.float32)
                return S, P, E

            S, P, E = step_out

            t = P - 1.0
            pl2 = jnp.float32(_LOG2C[-1])
            for coef in _LOG2C[-2::-1]:
                pl2 = coef + t * pl2
            ef = (E - 127 * nsteps).astype(jnp.float32)
            gathered = plsc.load_gather(buf, [rowoff + idx_vec])
            return acc_w + S + _LN2 * (ef + pl2) - ev_vec * gathered

        acc_w = lax.fori_loop(0, _CH // 16, group, acc_w)

    stage_v[...] = acc_w
    pltpu.sync_copy(stage_v, out_hbm.at[wid])


@jax.jit
def _run(phi, idx, ev):
    mesh = plsc.VectorSubcoreMesh(core_axis_name="c", subcore_axis_name="s")
    partials = pl.kernel(
        _sc_kernel,
        mesh=mesh,
        out_type=jax.ShapeDtypeStruct((_NW, 16), jnp.float32),
        scratch_types=[
            pltpu.VMEM((_CH * T,), jnp.float32),
            pltpu.VMEM((_CH * T,), jnp.float32),
            pltpu.VMEM((_RW,), jnp.int32),
            pltpu.VMEM((_RW,), jnp.float32),
            pltpu.VMEM((16,), jnp.float32),
            pltpu.SemaphoreType.DMA,
            pltpu.SemaphoreType.DMA,
        ],
        compiler_params=pltpu.CompilerParams(
            use_tc_tiling_on_sc=False, needs_layout_passes=False),
    )(phi.reshape(-1), idx, ev)
    return jnp.sum(partials) / B


def kernel(phi, idx_durations, events):
    return _run(phi, idx_durations.reshape(-1), events.reshape(-1))


# speedup vs baseline: 1.1911x; 1.0347x over previous
import jax
import jax.numpy as jnp
from jax import lax
from jax.experimental import pallas as pl
from jax.experimental.pallas import tpu as pltpu
from jax.experimental.pallas import tpu_sc as plsc

B = 16384
T = 512

_LOG2C = (
    5.6422440275483154e-08,
    1.442685851294528,
    -0.7210957682030537,
    0.4781764415123899,
    -0.34542933660333985,
    0.2380419836756127,
    -0.13314692748387624,
    0.04943336843736993,
    -0.008665699320087797,
)
_LN2 = 0.6931471805599453
_LOG2E = 1.4426950408889634

_EXP2C = (
    0.9999980397841518,
    -0.6930489339094811,
    0.23943060364123772,
    -0.05321311778871329,
    0.006835154726653606,
)

_INFO = plsc.get_sparse_core_info()
_NC = _INFO.num_cores
_NS = _INFO.num_subcores
_NW = _NC * _NS
_RW = B // _NW
_CH = 64
_NCHUNK = _RW // _CH


def _treemul(vals):
    while len(vals) > 1:
        vals = [a * b for a, b in zip(vals[::2], vals[1::2])]
    return vals[0]


def _treeadd(vals):
    while len(vals) > 1:
        vals = [a + b for a, b in zip(vals[::2], vals[1::2])]
    return vals[0]


def _sc_kernel(phi_hbm, idx_hbm, ev_hbm, out_hbm,
               buf0, buf1, idx_v, ev_v, stage_v, sem0, sem1):
    wid = lax.axis_index("s") * _NC + lax.axis_index("c")
    base = wid * _RW

    pltpu.sync_copy(idx_hbm.at[pl.ds(base, _RW)], idx_v)
    pltpu.sync_copy(ev_hbm.at[pl.ds(base, _RW)], ev_v)

    bufs = (buf0, buf1)
    sems = (sem0, sem1)

    def start(c):
        return pltpu.async_copy(
            phi_hbm.at[pl.ds(base + c * _CH, _CH), :],
            bufs[c % 2], sems[c % 2])

    lane = lax.iota(jnp.int32, 16)
    copies = [None] * _NCHUNK
    copies[0] = start(0)
    acc_w = jnp.zeros((16,), jnp.float32)

    for c in range(_NCHUNK):
        if c + 1 < _NCHUNK:
            copies[c + 1] = start(c + 1)
        copies[c].wait()
        buf = bufs[c % 2]

        def group(g, acc_w):
            rows16 = lane + g * 16
            idx_vec = idx_v[pl.ds(c * _CH + g * 16, 16)]
            ev_vec = ev_v[pl.ds(c * _CH + g * 16, 16)]
            nsteps = jnp.max(idx_vec) // 16 + 1
            colk = [(lane + k) & 15 for k in range(16)]

            init = (jnp.zeros((16,), jnp.float32),
                    jnp.ones((16,), jnp.float32),
                    jnp.zeros((16,), jnp.int32))

            @plsc.parallel_loop(0, nsteps, unroll=1, carry=init)
            def step_out(s, carry):
                S, P, E = carry
                s16 = s * 16
                rem = idx_vec - s16
                fs = []
                ss = []
                for k in range(16):
                    x = plsc.load_gather(buf, [rows16, s16 + colk[k]])
                    xm = jnp.where(rem >= colk[k], x, -1e30)
                    u = jnp.exp(-jnp.abs(xm))
                    fs.append(1.0 + u)
                    ss.append(jnp.maximum(xm, 0.0))
                P = P * _treemul(fs)
                S = S + _treeadd(ss)
                bits = lax.bitcast_convert_type(P, jnp.int32)
                E = E + lax.shift_right_logical(bits, 23)
                P = lax.bitcast_convert_type(
                    (bits & 0x007FFFFF) | 0x3F800000, jnp.float32)
                return S, P, E

            S, P, E = step_out

            t = P - 1.0
            pl2 = jnp.float32(_LOG2C[-1])
            for coef in _LOG2C[-2::-1]:
                pl2 = coef + t * pl2
            ef = (E - 127 * nsteps).astype(jnp.float32)
            gathered = plsc.load_gather(buf, [rows16, idx_vec])
            return acc_w + S + _LN2 * (ef + pl2) - ev_vec * gathered

        acc_w = lax.fori_loop(0, _CH // 16, group, acc_w)

    stage_v[...] = acc_w
    pltpu.sync_copy(stage_v, out_hbm.at[wid])


@jax.jit
def _run(phi, idx, ev):
    mesh = plsc.VectorSubcoreMesh(core_axis_name="c", subcore_axis_name="s")
    partials = pl.kernel(
        _sc_kernel,
        mesh=mesh,
        out_type=jax.ShapeDtypeStruct((_NW, 16), jnp.float32),
        scratch_types=[
            pltpu.VMEM((_CH, T), jnp.float32),
            pltpu.VMEM((_CH, T), jnp.float32),
            pltpu.VMEM((_RW,), jnp.int32),
            pltpu.VMEM((_RW,), jnp.float32),
            pltpu.VMEM((16,), jnp.float32),
            pltpu.SemaphoreType.DMA,
            pltpu.SemaphoreType.DMA,
        ],
        compiler_params=pltpu.CompilerParams(
            use_tc_tiling_on_sc=False, needs_layout_passes=False),
    )(phi, idx, ev)
    return jnp.sum(partials) / B


def kernel(phi, idx_durations, events):
    return _run(phi, idx_durations.reshape(-1), events.reshape(-1))


# speedup vs baseline: 1.4486x; 1.2161x over previous
import jax
import jax.numpy as jnp
from jax import lax
from jax.experimental import pallas as pl
from jax.experimental.pallas import tpu as pltpu
from jax.experimental.pallas import tpu_sc as plsc

B = 16384
T = 512

_LOG2C = (
    5.6422440275483154e-08,
    1.442685851294528,
    -0.7210957682030537,
    0.4781764415123899,
    -0.34542933660333985,
    0.2380419836756127,
    -0.13314692748387624,
    0.04943336843736993,
    -0.008665699320087797,
)
_LN2 = 0.6931471805599453
_LOG2E = 1.4426950408889634

_EXP2C = (
    0.9999980397841518,
    -0.6930489339094811,
    0.23943060364123772,
    -0.05321311778871329,
    0.006835154726653606,
)

_INFO = plsc.get_sparse_core_info()
_NC = _INFO.num_cores
_NS = _INFO.num_subcores
_NW = _NC * _NS
_RW = B // _NW
_CH = 64
_NCHUNK = _RW // _CH


def _treemul(vals):
    while len(vals) > 1:
        vals = [a * b for a, b in zip(vals[::2], vals[1::2])]
    return vals[0]


def _treeadd(vals):
    while len(vals) > 1:
        vals = [a + b for a, b in zip(vals[::2], vals[1::2])]
    return vals[0]


def _sc_kernel(phi_hbm, idx_hbm, ev_hbm, out_hbm,
               buf0, buf1, idx_v, ev_v, stage_v, sem0, sem1):
    wid = lax.axis_index("s") * _NC + lax.axis_index("c")
    base = wid * _RW

    pltpu.sync_copy(idx_hbm.at[pl.ds(base, _RW)], idx_v)
    pltpu.sync_copy(ev_hbm.at[pl.ds(base, _RW)], ev_v)

    bufs = (buf0, buf1)
    sems = (sem0, sem1)

    def start(c):
        return pltpu.async_copy(
            phi_hbm.at[pl.ds(base + c * _CH, _CH), :],
            bufs[c % 2], sems[c % 2])

    lane = lax.iota(jnp.int32, 16)
    copies = [None] * _NCHUNK
    copies[0] = start(0)
    acc_w = jnp.zeros((16,), jnp.float32)

    for c in range(_NCHUNK):
        if c + 1 < _NCHUNK:
            copies[c + 1] = start(c + 1)
        copies[c].wait()
        buf = bufs[c % 2]

        def group(g, acc_w):
            rows16 = lane + g * 16
            idx_vec = idx_v[pl.ds(c * _CH + g * 16, 16)]
            ev_vec = ev_v[pl.ds(c * _CH + g * 16, 16)]
            nsteps = jnp.max(idx_vec) // 16 + 1
            colk = [(lane + k) & 15 for k in range(16)]

            init = (jnp.zeros((16,), jnp.float32),
                    jnp.ones((16,), jnp.float32),
                    jnp.zeros((16,), jnp.int32))

            @plsc.parallel_loop(0, nsteps, unroll=1, carry=init)
            def step_out(s, carry):
                S, P, E = carry
                s16 = s * 16
                rem = idx_vec - s16
                fs = []
                ss = []
                for k in range(16):
                    x = plsc.load_gather(buf, [rows16, s16 + colk[k]])
                    xm = jnp.where(rem >= colk[k], x, -1e30)
                    u = jnp.exp(-jnp.abs(xm))
                    fs.append(1.0 + u)
                    ss.append(jnp.maximum(xm, 0.0))
                P = P * _treemul(fs)
                S = S + _treeadd(ss)
                bits = lax.bitcast_convert_type(P, jnp.int32)
                E = E + lax.shift_right_logical(bits, 23)
                P = lax.bitcast_convert_type(
                    (bits & 0x007FFFFF) | 0x3F800000, jnp.float32)
                return S, P, E

            S, P, E = step_out

            t = P - 1.0
            pl2 = jnp.float32(_LOG2C[-1])
            for coef in _LOG2C[-2::-1]:
                pl2 = coef + t * pl2
            ef = (E - 127 * nsteps).astype(jnp.float32)
            gathered = plsc.load_gather(buf, [rows16, idx_vec])
            return acc_w + S + _LN2 * (ef + pl2) - ev_vec * gathered

        acc_w = lax.fori_loop(0, _CH // 16, group, acc_w)

    stage_v[...] = acc_w
    pltpu.sync_copy(stage_v, out_hbm.at[wid])


@jax.jit
def _run(phi, idx, ev):
    mesh = plsc.VectorSubcoreMesh(core_axis_name="c", subcore_axis_name="s")
    partials = pl.kernel(
        _sc_kernel,
        mesh=mesh,
        out_type=jax.ShapeDtypeStruct((_NW, 16), jnp.float32),
        scratch_types=[
            pltpu.VMEM((_CH, T), jnp.float32),
            pltpu.VMEM((_CH, T), jnp.float32),
            pltpu.VMEM((_RW,), jnp.int32),
            pltpu.VMEM((_RW,), jnp.float32),
            pltpu.VMEM((16,), jnp.float32),
            pltpu.SemaphoreType.DMA,
            pltpu.SemaphoreType.DMA,
        ],
        compiler_params=pltpu.CompilerParams(
            use_tc_tiling_on_sc=True, needs_layout_passes=False),
    )(phi, idx, ev)
    return jnp.sum(partials) / B


def kernel(phi, idx_durations, events):
    return _run(phi, idx_durations.reshape(-1), events.reshape(-1))


# speedup vs baseline: 1.7796x; 1.2285x over previous
import jax
import jax.numpy as jnp
from jax import lax
from jax.experimental import pallas as pl
from jax.experimental.pallas import tpu as pltpu
from jax.experimental.pallas import tpu_sc as plsc

B = 16384
T = 512

_LOG2C = (
    5.6422440275483154e-08,
    1.442685851294528,
    -0.7210957682030537,
    0.4781764415123899,
    -0.34542933660333985,
    0.2380419836756127,
    -0.13314692748387624,
    0.04943336843736993,
    -0.008665699320087797,
)
_LN2 = 0.6931471805599453
_LOG2E = 1.4426950408889634

_EXP2C = (
    0.9999980397841518,
    -0.6930489339094811,
    0.23943060364123772,
    -0.05321311778871329,
    0.006835154726653606,
)

_INFO = plsc.get_sparse_core_info()
_NC = _INFO.num_cores
_NS = _INFO.num_subcores
_NW = _NC * _NS
_RW = B // _NW
_CH = 64
_NCHUNK = _RW // _CH


def _treemul(vals):
    while len(vals) > 1:
        vals = [a * b for a, b in zip(vals[::2], vals[1::2])]
    return vals[0]


def _treeadd(vals):
    while len(vals) > 1:
        vals = [a + b for a, b in zip(vals[::2], vals[1::2])]
    return vals[0]


def _sc_kernel(phi_hbm, idx_hbm, ev_hbm, out_hbm,
               buf0, buf1, idx_v, ev_v, stage_v, counts_v, offs_v, perm_v,
               sem0, sem1):
    wid = lax.axis_index("s") * _NC + lax.axis_index("c")
    base = wid * _RW

    pltpu.sync_copy(idx_hbm.at[pl.ds(base, _RW)], idx_v)
    pltpu.sync_copy(ev_hbm.at[pl.ds(base, _RW)], ev_v)

    bufs = (buf0, buf1)
    sems = (sem0, sem1)

    def start(c):
        return pltpu.async_copy(
            phi_hbm.at[pl.ds(base + c * _CH, _CH), :],
            bufs[c % 2], sems[c % 2])

    lane = lax.iota(jnp.int32, 16)
    copies = [None] * _NCHUNK
    copies[0] = start(0)
    acc_w = jnp.zeros((16,), jnp.float32)

    for c in range(_NCHUNK):
        if c + 1 < _NCHUNK:
            copies[c + 1] = start(c + 1)
        copies[c].wait()
        buf = bufs[c % 2]

        zeros16 = jnp.zeros((16,), jnp.int32)
        ones16 = jnp.full((16,), 1, jnp.int32)
        counts_v[pl.ds(0, 16)] = zeros16
        counts_v[pl.ds(16, 16)] = zeros16
        for s in range(_CH // 16):
            b = lax.shift_right_logical(
                idx_v[pl.ds(c * _CH + s * 16, 16)], 4)
            plsc.addupdate_scatter(counts_v, [b], ones16)
        c0 = counts_v[pl.ds(0, 16)]
        c1 = counts_v[pl.ds(16, 16)]
        cum0 = plsc.cumsum(c0)
        cum1 = plsc.cumsum(c1)
        offs_v[pl.ds(0, 16)] = cum0 - c0
        offs_v[pl.ds(16, 16)] = cum1 - c1 + jnp.max(cum0)
        for s in range(_CH // 16):
            perm_v[pl.ds(s * 16, 16)] = zeros16
        for s in range(_CH // 16):
            b = lax.shift_right_logical(
                idx_v[pl.ds(c * _CH + s * 16, 16)], 4)
            bases = plsc.load_gather(offs_v, [b])
            rank, _ = plsc.scan_count(b)
            pos = jnp.clip(bases + rank - 1, 0, _CH - 1)
            plsc.store_scatter(perm_v, [pos], lane + s * 16)
            plsc.addupdate_scatter(offs_v, [b], ones16)

        def group(g, acc_w):
            rows16 = perm_v[pl.ds(g * 16, 16)]
            idx_vec = plsc.load_gather(idx_v, [c * _CH + rows16])
            ev_vec = plsc.load_gather(ev_v, [c * _CH + rows16])
            nsteps = jnp.max(idx_vec) // 16 + 1
            colk = [(lane + k) & 15 for k in range(16)]

            init = (jnp.zeros((16,), jnp.float32),
                    jnp.ones((16,), jnp.float32),
                    jnp.zeros((16,), jnp.int32))

            @plsc.parallel_loop(0, nsteps, unroll=1, carry=init)
            def step_out(s, carry):
                S, P, E = carry
                s16 = s * 16
                rem = idx_vec - s16
                fs = []
                ss = []
                for k in range(16):
                    x = plsc.load_gather(buf, [rows16, s16 + colk[k]])
                    xm = jnp.where(rem >= colk[k], x, -1e30)
                    u = jnp.exp(-jnp.abs(xm))
                    fs.append(1.0 + u)
                    ss.append(jnp.maximum(xm, 0.0))
                P = P * _treemul(fs)
                S = S + _treeadd(ss)
                bits = lax.bitcast_convert_type(P, jnp.int32)
                E = E + lax.shift_right_logical(bits, 23)
                P = lax.bitcast_convert_type(
                    (bits & 0x007FFFFF) | 0x3F800000, jnp.float32)
                return S, P, E

            S, P, E = step_out

            t = P - 1.0
            pl2 = jnp.float32(_LOG2C[-1])
            for coef in _LOG2C[-2::-1]:
                pl2 = coef + t * pl2
            ef = (E - 127 * nsteps).astype(jnp.float32)
            gathered = plsc.load_gather(buf, [rows16, idx_vec])
            return acc_w + S + _LN2 * (ef + pl2) - ev_vec * gathered

        acc_w = lax.fori_loop(0, _CH // 16, group, acc_w)

    stage_v[...] = acc_w
    pltpu.sync_copy(stage_v, out_hbm.at[wid])


@jax.jit
def _run(phi, idx, ev):
    mesh = plsc.VectorSubcoreMesh(core_axis_name="c", subcore_axis_name="s")
    partials = pl.kernel(
        _sc_kernel,
        mesh=mesh,
        out_type=jax.ShapeDtypeStruct((_NW, 16), jnp.float32),
        scratch_types=[
            pltpu.VMEM((_CH, T), jnp.float32),
            pltpu.VMEM((_CH, T), jnp.float32),
            pltpu.VMEM((_RW,), jnp.int32),
            pltpu.VMEM((_RW,), jnp.float32),
            pltpu.VMEM((16,), jnp.float32),
            pltpu.VMEM((32,), jnp.int32),
            pltpu.VMEM((32,), jnp.int32),
            pltpu.VMEM((_CH,), jnp.int32),
            pltpu.SemaphoreType.DMA,
            pltpu.SemaphoreType.DMA,
        ],
        compiler_params=pltpu.CompilerParams(
            use_tc_tiling_on_sc=True, needs_layout_passes=False),
    )(phi, idx, ev)
    return jnp.sum(partials) / B


def kernel(phi, idx_durations, events):
    return _run(phi, idx_durations.reshape(-1), events.reshape(-1))
